# baseline (speedup 1.0000x reference)
import functools

import jax
import jax.numpy as jnp
from jax import lax
from jax.experimental import pallas as pl
from jax.experimental.pallas import tpu as pltpu
from jax.experimental.pallas import tpu_sc as plsc

NC = 2
NS = 16
L = 16
NW = NC * NS

B = 16384
D = 64
USER_N = 100000
ITEM_N = 1000000
U_BANDS = (USER_N + 127) // 128
I_BANDS = (ITEM_N + 127) // 128
U_PB = -(-U_BANDS // NW)
I_PB = -(-I_BANDS // NW)
U_PAD = U_BANDS * 128
CHUNK = 4096
HMAX = 4096
WIN = 3
NWIN = -(-I_PB // WIN)
PMAX = 128
NGR = PMAX // L
PAD = 2 * B

_CP = pltpu.CompilerParams(needs_layout_passes=False,
                           use_tc_tiling_on_sc=True)


def _detile_user_body(ut_hbm, ustage_hbm, bb, st, sem, sem2):
    wid = lax.axis_index("s") * NC + lax.axis_index("c")
    lo = wid * U_PB
    hi = jnp.minimum(lo + U_PB, U_BANDS)
    lane = lax.iota(jnp.int32, L)
    n = U_PB

    def fetch(i, s):
        c = jnp.minimum(lo + i, hi - 1)
        off = pl.multiple_of(c * 128, 128)
        pltpu.async_copy(ut_hbm.at[:, pl.ds(off, 128)], bb.at[s], sem)

    fetch(0, 0)

    def band_body(i, _):
        s = i & 1

        @pl.when(i + 1 < n)
        def _():
            fetch(i + 1, 1 - s)

        pltpu.make_async_copy(ut_hbm.at[:, pl.ds(0, 128)], bb.at[s],
                              sem).wait()
        for d in range(D):
            col = jnp.full((L,), d, jnp.int32)
            sv = jnp.full((L,), s, jnp.int32)
            for p in range(8):
                v = bb[s, d, pl.ds(p * L, L)]
                plsc.store_scatter(st, [sv, p * L + lane, col], v)

        @pl.when(i > 0)
        def _():
            pltpu.make_async_copy(st.at[1 - s],
                                  ustage_hbm.at[pl.ds(0, 128), :],
                                  sem2).wait()

        c = jnp.minimum(lo + i, hi - 1)
        off = pl.multiple_of(c * 128, 128)
        pltpu.async_copy(st.at[s], ustage_hbm.at[pl.ds(off, 128), :], sem2)
        return 0

    lax.fori_loop(0, n, band_body, 0)
    pltpu.make_async_copy(st.at[(n - 1) & 1],
                          ustage_hbm.at[pl.ds(0, 128), :], sem2).wait()


def _item_body(user_hbm, item_i_hbm, item_j_hbm, it_hbm, ustage_hbm,
               pred_hbm,
               uid_all, ichunk, hb, hidx, pb, pcol, uidb, bb, urow, res,
               sem, sem2):
    wid = lax.axis_index("s") * NC + lax.axis_index("c")
    lo = wid * I_PB
    hi = jnp.minimum(lo + I_PB, I_BANDS)
    lane = lax.iota(jnp.int32, L)

    pltpu.sync_copy(user_hbm, uid_all)

    def scan_stream(src_hbm, boff, ptr0):
        pltpu.sync_copy(src_hbm, ichunk)

        def chunk_body(k, ptr):
            def vec_body(q, ptr):
                iv = ichunk[pl.ds(k * CHUNK + q * L, L)]
                band = iv >> 7
                m = (band >= lo) & (band < hi)
                bv = boff + k * CHUNK + q * L + lane
                pc = jnp.minimum(ptr, HMAX)
                plsc.store_compressed(hb.at[pl.ds(pc, L)], bv, mask=m)
                plsc.store_compressed(hidx.at[pl.ds(pc, L)], iv, mask=m)
                return ptr + plsc.all_reduce_population_count(m)[0]

            return lax.fori_loop(0, CHUNK // L, vec_body, ptr)

        return lax.fori_loop(0, B // CHUNK, chunk_body, ptr0)

    ptr = scan_stream(item_i_hbm, 0, jnp.int32(0))
    ptr = scan_stream(item_j_hbm, B, ptr)
    ptr = jnp.minimum(ptr, HMAX)
    nvec = (ptr + L - 1) // L

    def rescan(w, s):
        c0 = lo + w * WIN
        cend = jnp.minimum(c0 + WIN, hi)
        cb = jnp.minimum(c0, I_BANDS - WIN)
        padv = jnp.full((L,), PAD, jnp.int32)
        for g in range(NGR):
            pb[s, pl.ds(g * L, L)] = padv
            pcol[s, pl.ds(g * L, L)] = jnp.zeros((L,), jnp.int32)

        def pack_body(q, p2):
            ok = (q * L + lane) < ptr
            iv = hidx[pl.ds(q * L, L)]
            bv = hb[pl.ds(q * L, L)]
            band = iv >> 7
            m = ok & (band >= c0) & (band < cend)
            scol = (band - cb) * 128 + (iv & 127)
            p2c = jnp.minimum(p2, PMAX - L)
            plsc.store_compressed(pb.at[s, pl.ds(p2c, L)], bv, mask=m)
            plsc.store_compressed(pcol.at[s, pl.ds(p2c, L)], scol, mask=m)
            return p2 + plsc.all_reduce_population_count(m)[0]

        p2 = jnp.minimum(lax.fori_loop(0, nvec, pack_body, jnp.int32(0)),
                         PMAX)
        for g in range(NGR):
            ev = pb[s, pl.ds(g * L, L)]
            uid = plsc.load_gather(uid_all, [ev & (B - 1)])
            uidb[s, pl.ds(g * L, L)] = uid
        return p2

    def fire(w, s, p2):
        cb = jnp.minimum(jnp.minimum(lo + w * WIN, I_BANDS - WIN), hi - 1)
        off = pl.multiple_of(cb * 128, 128)
        pltpu.async_copy(it_hbm.at[:, pl.ds(off, WIN * 128)], bb.at[s], sem)
        pltpu.async_copy(ustage_hbm.at[uidb.at[s]], urow.at[s], sem)

    p2_0 = rescan(0, 0)
    fire(0, 0, p2_0)

    def window_body(w, carry):
        p2w, p2prev = carry
        s = w & 1
        s2 = 1 - s

        @pl.when(w > 0)
        def _():
            pltpu.make_async_copy(res.at[s2],
                                  pred_hbm.at[pl.ds(0, PMAX)], sem2).wait()

        p2n = rescan(w + 1, s2)

        @pl.when(w + 1 < NWIN)
        def _():
            fire(w + 1, s2, p2n)

        pltpu.make_async_copy(it_hbm.at[:, pl.ds(0, WIN * 128)],
                              bb.at[s], sem).wait()
        pltpu.make_async_copy(ustage_hbm.at[pl.ds(0, PMAX), :],
                              urow.at[s], sem).wait()
        ngw = (p2w + L - 1) // L

        sv = jnp.full((L,), s, jnp.int32)
        for g in range(NGR):
            @pl.when(g < ngw)
            def _():
                scol = pcol[s, pl.ds(g * L, L)]
                acc = jnp.zeros((L,), jnp.float32)
                for d in range(D):
                    dv = jnp.full((L,), d, jnp.int32)
                    iv_d = plsc.load_gather(bb, [sv, dv, scol])
                    u_d = plsc.load_gather(urow, [sv, g * L + lane, dv])
                    acc = acc + iv_d * u_d
                res[s, pl.ds(g * L, L)] = acc

        pltpu.async_copy(res.at[s], pred_hbm.at[pb.at[s]], sem2)

        return (p2n, p2w)

    lax.fori_loop(0, NWIN, window_body, (p2_0, jnp.int32(0)))
    pltpu.make_async_copy(res.at[(NWIN - 1) & 1],
                          pred_hbm.at[pl.ds(0, PMAX)], sem2).wait()


@jax.jit
def _bpr(user, item_i, item_j, embed_user_weight, embed_item_weight):
    mesh = plsc.VectorSubcoreMesh(core_axis_name="c", subcore_axis_name="s",
                                  num_cores=NC, num_subcores=NS)
    k1 = functools.partial(
        pl.kernel,
        out_type=jax.ShapeDtypeStruct((U_PAD, 128), jnp.float32),
        mesh=mesh,
        compiler_params=_CP,
        scratch_types=[
            pltpu.VMEM((2, D, 128), jnp.float32),
            pltpu.VMEM((2, 128, 128), jnp.float32),
            pltpu.SemaphoreType.DMA,
            pltpu.SemaphoreType.DMA,
        ],
    )(_detile_user_body)
    ustage = k1(embed_user_weight.T)

    k2 = functools.partial(
        pl.kernel,
        out_type=jax.ShapeDtypeStruct((PAD + L,), jnp.float32),
        mesh=mesh,
        compiler_params=_CP,
        scratch_types=[
            pltpu.VMEM((B,), jnp.int32),
            pltpu.VMEM((B,), jnp.int32),
            pltpu.VMEM((HMAX + L,), jnp.int32),
            pltpu.VMEM((HMAX + L,), jnp.int32),
            pltpu.VMEM((2, PMAX), jnp.int32),
            pltpu.VMEM((2, PMAX), jnp.int32),
            pltpu.VMEM((2, PMAX), jnp.int32),
            pltpu.VMEM((2, D, WIN * 128), jnp.float32),
            pltpu.VMEM((2, PMAX, 128), jnp.float32),
            pltpu.VMEM((2, PMAX), jnp.float32),
            pltpu.SemaphoreType.DMA,
            pltpu.SemaphoreType.DMA,
        ],
    )(_item_body)
    pred = k2(user, item_i, item_j, embed_item_weight.T, ustage)
    return pred[:B], pred[B:PAD]


def kernel(user, item_i, item_j, embed_user_weight, embed_item_weight):
    return _bpr(user, item_i, item_j, embed_user_weight, embed_item_weight)


# speedup vs baseline: 15.6715x; 15.6715x over previous
import functools

import jax
import jax.numpy as jnp
from jax import lax
from jax.experimental import pallas as pl
from jax.experimental.pallas import tpu as pltpu
from jax.experimental.pallas import tpu_sc as plsc

NC = 2
NS = 16
L = 16
NW = NC * NS

B = 16384
D = 64
USER_N = 100000
ITEM_N = 1000000
U_BANDS = (USER_N + 127) // 128
I_BANDS = (ITEM_N + 127) // 128
U_PB = -(-U_BANDS // NW)
I_PB = -(-I_BANDS // NW)
U_PAD = U_BANDS * 128
CHUNK = 4096
HMAX = 4096
WIN = 3
NWIN = -(-I_PB // WIN)
PMAX = 128
NGR = PMAX // L
PAD = 2 * B
OUT_N = PAD + NW * PMAX

_CP = pltpu.CompilerParams(needs_layout_passes=False,
                           use_tc_tiling_on_sc=True)


def _detile_user_body(ut_hbm, ustage_hbm, bb, st, sem, sem2):
    wid = lax.axis_index("s") * NC + lax.axis_index("c")
    lo = wid * U_PB
    hi = jnp.minimum(lo + U_PB, U_BANDS)
    lane = lax.iota(jnp.int32, L)
    n = U_PB

    def fetch(i, s):
        c = jnp.minimum(lo + i, hi - 1)
        off = pl.multiple_of(c * 128, 128)
        pltpu.async_copy(ut_hbm.at[:, pl.ds(off, 128)], bb.at[s], sem)

    fetch(0, 0)

    def band_body(i, _):
        s = i & 1

        @pl.when(i + 1 < n)
        def _():
            fetch(i + 1, 1 - s)

        pltpu.make_async_copy(ut_hbm.at[:, pl.ds(0, 128)], bb.at[s],
                              sem).wait()
        for d in range(D):
            col = jnp.full((L,), d, jnp.int32)
            sv = jnp.full((L,), s, jnp.int32)
            for p in range(8):
                v = bb[s, d, pl.ds(p * L, L)]
                plsc.store_scatter(st, [sv, p * L + lane, col], v)

        @pl.when(i > 0)
        def _():
            pltpu.make_async_copy(st.at[1 - s],
                                  ustage_hbm.at[pl.ds(0, 128), :],
                                  sem2).wait()

        c = jnp.minimum(lo + i, hi - 1)
        off = pl.multiple_of(c * 128, 128)
        pltpu.async_copy(st.at[s], ustage_hbm.at[pl.ds(off, 128), :], sem2)
        return 0

    lax.fori_loop(0, n, band_body, 0)
    pltpu.make_async_copy(st.at[(n - 1) & 1],
                          ustage_hbm.at[pl.ds(0, 128), :], sem2).wait()


def _item_body(user_hbm, item_i_hbm, item_j_hbm, it_hbm, ustage_hbm,
               pred_hbm,
               uid_all, ichunk, hb, hidx, pb, pcol, uidb, bb, urow, res,
               sem, sem2):
    wid = lax.axis_index("s") * NC + lax.axis_index("c")
    lo = wid * I_PB
    hi = jnp.minimum(lo + I_PB, I_BANDS)
    lane = lax.iota(jnp.int32, L)

    pltpu.sync_copy(user_hbm, uid_all)

    def scan_stream(src_hbm, boff, ptr0):
        pltpu.sync_copy(src_hbm, ichunk)

        def chunk_body(k, ptr):
            def vec_body(q, ptr):
                iv = ichunk[pl.ds(k * CHUNK + q * L, L)]
                band = iv >> 7
                m = (band >= lo) & (band < hi)
                bv = boff + k * CHUNK + q * L + lane
                pc = jnp.minimum(ptr, HMAX)
                plsc.store_compressed(hb.at[pl.ds(pc, L)], bv, mask=m)
                plsc.store_compressed(hidx.at[pl.ds(pc, L)], iv, mask=m)
                return ptr + plsc.all_reduce_population_count(m)[0]

            return lax.fori_loop(0, CHUNK // L, vec_body, ptr)

        return lax.fori_loop(0, B // CHUNK, chunk_body, ptr0)

    ptr = scan_stream(item_i_hbm, 0, jnp.int32(0))
    ptr = scan_stream(item_j_hbm, B, ptr)
    ptr = jnp.minimum(ptr, HMAX)
    nvec = (ptr + L - 1) // L

    def rescan(w, s):
        c0 = lo + w * WIN
        cend = jnp.minimum(c0 + WIN, hi)
        cb = jnp.minimum(c0, I_BANDS - WIN)
        for g in range(NGR):
            pb[s, pl.ds(g * L, L)] = PAD + wid * PMAX + g * L + lane
            pcol[s, pl.ds(g * L, L)] = jnp.zeros((L,), jnp.int32)

        def pack_body(q, p2):
            ok = (q * L + lane) < ptr
            iv = hidx[pl.ds(q * L, L)]
            bv = hb[pl.ds(q * L, L)]
            band = iv >> 7
            m = ok & (band >= c0) & (band < cend)
            scol = (band - cb) * 128 + (iv & 127)
            p2c = jnp.minimum(p2, PMAX - L)
            plsc.store_compressed(pb.at[s, pl.ds(p2c, L)], bv, mask=m)
            plsc.store_compressed(pcol.at[s, pl.ds(p2c, L)], scol, mask=m)
            return p2 + plsc.all_reduce_population_count(m)[0]

        p2 = jnp.minimum(lax.fori_loop(0, nvec, pack_body, jnp.int32(0)),
                         PMAX)
        for g in range(NGR):
            ev = pb[s, pl.ds(g * L, L)]
            uid = plsc.load_gather(uid_all, [ev & (B - 1)])
            uidb[s, pl.ds(g * L, L)] = uid
        return p2

    def fire(w, s, p2):
        cb = jnp.minimum(jnp.minimum(lo + w * WIN, I_BANDS - WIN), hi - 1)
        off = pl.multiple_of(cb * 128, 128)
        pltpu.async_copy(it_hbm.at[:, pl.ds(off, WIN * 128)], bb.at[s], sem)
        pltpu.async_copy(ustage_hbm.at[uidb.at[s]], urow.at[s], sem)

    p2_0 = rescan(0, 0)
    fire(0, 0, p2_0)

    def window_body(w, carry):
        p2w, p2prev = carry
        s = w & 1
        s2 = 1 - s

        @pl.when(w > 0)
        def _():
            pltpu.make_async_copy(res.at[s2],
                                  pred_hbm.at[pl.ds(0, PMAX)], sem2).wait()

        p2n = rescan(w + 1, s2)

        @pl.when(w + 1 < NWIN)
        def _():
            fire(w + 1, s2, p2n)

        pltpu.make_async_copy(it_hbm.at[:, pl.ds(0, WIN * 128)],
                              bb.at[s], sem).wait()
        pltpu.make_async_copy(ustage_hbm.at[pl.ds(0, PMAX), :],
                              urow.at[s], sem).wait()
        ngw = (p2w + L - 1) // L

        sv = jnp.full((L,), s, jnp.int32)
        for g in range(NGR):
            @pl.when(g < ngw)
            def _():
                scol = pcol[s, pl.ds(g * L, L)]
                acc = jnp.zeros((L,), jnp.float32)
                for d in range(D):
                    dv = jnp.full((L,), d, jnp.int32)
                    iv_d = plsc.load_gather(bb, [sv, dv, scol])
                    u_d = plsc.load_gather(urow, [sv, g * L + lane, dv])
                    acc = acc + iv_d * u_d
                res[s, pl.ds(g * L, L)] = acc

        pltpu.async_copy(res.at[s], pred_hbm.at[pb.at[s]], sem2)

        return (p2n, p2w)

    lax.fori_loop(0, NWIN, window_body, (p2_0, jnp.int32(0)))
    pltpu.make_async_copy(res.at[(NWIN - 1) & 1],
                          pred_hbm.at[pl.ds(0, PMAX)], sem2).wait()


@jax.jit
def _bpr(user, item_i, item_j, embed_user_weight, embed_item_weight):
    mesh = plsc.VectorSubcoreMesh(core_axis_name="c", subcore_axis_name="s",
                                  num_cores=NC, num_subcores=NS)
    k1 = functools.partial(
        pl.kernel,
        out_type=jax.ShapeDtypeStruct((U_PAD, 128), jnp.float32),
        mesh=mesh,
        compiler_params=_CP,
        scratch_types=[
            pltpu.VMEM((2, D, 128), jnp.float32),
            pltpu.VMEM((2, 128, 128), jnp.float32),
            pltpu.SemaphoreType.DMA,
            pltpu.SemaphoreType.DMA,
        ],
    )(_detile_user_body)
    ustage = k1(embed_user_weight.T)

    k2 = functools.partial(
        pl.kernel,
        out_type=jax.ShapeDtypeStruct((OUT_N,), jnp.float32),
        mesh=mesh,
        compiler_params=_CP,
        scratch_types=[
            pltpu.VMEM((B,), jnp.int32),
            pltpu.VMEM((B,), jnp.int32),
            pltpu.VMEM((HMAX + L,), jnp.int32),
            pltpu.VMEM((HMAX + L,), jnp.int32),
            pltpu.VMEM((2, PMAX), jnp.int32),
            pltpu.VMEM((2, PMAX), jnp.int32),
            pltpu.VMEM((2, PMAX), jnp.int32),
            pltpu.VMEM((2, D, WIN * 128), jnp.float32),
            pltpu.VMEM((2, PMAX, 128), jnp.float32),
            pltpu.VMEM((2, PMAX), jnp.float32),
            pltpu.SemaphoreType.DMA,
            pltpu.SemaphoreType.DMA,
        ],
    )(_item_body)
    pred = k2(user, item_i, item_j, embed_item_weight.T, ustage)
    return pred[:B], pred[B:PAD]


def kernel(user, item_i, item_j, embed_user_weight, embed_item_weight):
    return _bpr(user, item_i, item_j, embed_user_weight, embed_item_weight)


# speedup vs baseline: 74.2519x; 4.7380x over previous
import functools

import jax
import jax.numpy as jnp
from jax import lax
from jax.experimental import pallas as pl
from jax.experimental.pallas import tpu as pltpu
from jax.experimental.pallas import tpu_sc as plsc

NC = 2
NS = 16
L = 16
NW = NC * NS

B = 16384
D = 64
DP = 128
BPW = B // NW
HALF = BPW // 2
CHUNK = 128
NCHUNK = HALF // CHUNK


def _bpr_body(user_hbm, item_i_hbm, item_j_hbm, euw_hbm, eiw_hbm,
              out_i_hbm, out_j_hbm,
              u_idx, i_idx, j_idx, u_rows, vi_rows, vj_rows,
              tile_i, tile_j, pred_i, pred_j, sem):
    wid = lax.axis_index("s") * NC + lax.axis_index("c")
    base = wid * BPW
    lane_iota = lax.iota(jnp.int32, L)

    for c in range(2 * NCHUNK):
        off = base + c * CHUNK
        pltpu.sync_copy(user_hbm.at[pl.ds(off, CHUNK)], u_idx.at[c])
        pltpu.sync_copy(item_i_hbm.at[pl.ds(off, CHUNK)], i_idx.at[c])
        pltpu.sync_copy(item_j_hbm.at[pl.ds(off, CHUNK)], j_idx.at[c])

    for h in range(2):
        descs = []
        for c in range(NCHUNK):
            dst = pl.ds(c * CHUNK, CHUNK)
            cc = h * NCHUNK + c
            descs.append(pltpu.async_copy(
                euw_hbm.at[u_idx.at[cc]], u_rows.at[dst], sem))
            descs.append(pltpu.async_copy(
                eiw_hbm.at[i_idx.at[cc]], vi_rows.at[dst], sem))
            descs.append(pltpu.async_copy(
                eiw_hbm.at[j_idx.at[cc]], vj_rows.at[dst], sem))
        for dsc in descs:
            dsc.wait()

        def group_body(g, _):
            base_r = g * L
            for rr in range(L):
                r = base_r + rr
                acc_i = jnp.zeros((L,), jnp.float32)
                acc_j = jnp.zeros((L,), jnp.float32)
                for k in range(D // L):
                    sl = pl.ds(k * L, L)
                    u = u_rows[r, sl]
                    acc_i = acc_i + u * vi_rows[r, sl]
                    acc_j = acc_j + u * vj_rows[r, sl]
                col = lane_iota * L + rr
                plsc.store_scatter(tile_i, [col], acc_i)
                plsc.store_scatter(tile_j, [col], acc_j)
            vec_i = tile_i[pl.ds(0, L)]
            vec_j = tile_j[pl.ds(0, L)]
            for k in range(1, L):
                vec_i = vec_i + tile_i[pl.ds(k * L, L)]
                vec_j = vec_j + tile_j[pl.ds(k * L, L)]
            pred_i[pl.ds(h * HALF + base_r, L)] = vec_i
            pred_j[pl.ds(h * HALF + base_r, L)] = vec_j
            return 0

        lax.fori_loop(0, HALF // L, group_body, 0)

    pltpu.sync_copy(pred_i, out_i_hbm.at[pl.ds(base, BPW)])
    pltpu.sync_copy(pred_j, out_j_hbm.at[pl.ds(base, BPW)])


@jax.jit
def _bpr(user, item_i, item_j, embed_user_weight, embed_item_weight):
    mesh = plsc.VectorSubcoreMesh(core_axis_name="c", subcore_axis_name="s",
                                  num_cores=NC, num_subcores=NS)
    euw = jnp.pad(embed_user_weight, ((0, 0), (0, DP - D)))
    eiw = jnp.pad(embed_item_weight, ((0, 0), (0, DP - D)))
    f = functools.partial(
        pl.kernel,
        out_type=(jax.ShapeDtypeStruct((B,), jnp.float32),
                  jax.ShapeDtypeStruct((B,), jnp.float32)),
        mesh=mesh,
        compiler_params=pltpu.CompilerParams(needs_layout_passes=False,
                                             use_tc_tiling_on_sc=True),
        scratch_types=[
            pltpu.VMEM((2 * NCHUNK, CHUNK), jnp.int32),
            pltpu.VMEM((2 * NCHUNK, CHUNK), jnp.int32),
            pltpu.VMEM((2 * NCHUNK, CHUNK), jnp.int32),
            pltpu.VMEM((HALF, DP), jnp.float32),
            pltpu.VMEM((HALF, DP), jnp.float32),
            pltpu.VMEM((HALF, DP), jnp.float32),
            pltpu.VMEM((L * L,), jnp.float32),
            pltpu.VMEM((L * L,), jnp.float32),
            pltpu.VMEM((BPW,), jnp.float32),
            pltpu.VMEM((BPW,), jnp.float32),
            pltpu.SemaphoreType.DMA,
        ],
    )(_bpr_body)
    return f(user, item_i, item_j, euw, eiw)


def kernel(user, item_i, item_j, embed_user_weight, embed_item_weight):
    return _bpr(user, item_i, item_j, embed_user_weight, embed_item_weight)
